# baseline (speedup 1.0000x reference)
import functools

import jax
import jax.numpy as jnp
from jax import lax
from jax.experimental import pallas as pl
from jax.experimental.pallas import tpu as pltpu
from jax.experimental.pallas import tpu_sc as plsc

EMBED_DIM = 64
MARGIN = 0.1
BN_EPS = 1e-5
B = 16384

NC, NS = 2, 16
NW = NC * NS
ROWS_W = B // NW
CHUNK = 128
NCH = ROWS_W // CHUNK

N_CE = 9
N_RE = 2
N_SLOT = N_CE + N_RE
N_CR = 8

BLK = 2048
NBLK = B // BLK


def _sc_gather(ce_raw, pack1k, re_pack, cr_flat, idx_pair_w, idx_cr_w):
    mesh = plsc.VectorSubcoreMesh(core_axis_name="c", subcore_axis_name="s")

    @functools.partial(
        pl.kernel,
        out_type=(
            jax.ShapeDtypeStruct((N_SLOT * B, CHUNK), jnp.float32),
            jax.ShapeDtypeStruct((N_CR * B,), jnp.float32),
        ),
        mesh=mesh,
        scratch_types=[
            pltpu.VMEM((N_SLOT * ROWS_W,), jnp.int32),
            pltpu.VMEM((N_CR * ROWS_W,), jnp.int32),
            pltpu.VMEM((ROWS_W // 2, CHUNK), jnp.float32),
            pltpu.VMEM((ROWS_W // 2, CHUNK), jnp.float32),
            pltpu.VMEM((ROWS_W,), jnp.float32),
            pltpu.VMEM((16, 8, EMBED_DIM), jnp.float32),
            pltpu.SemaphoreType.DMA,
            pltpu.SemaphoreType.DMA,
            pltpu.SemaphoreType.DMA,
            pltpu.SemaphoreType.DMA,
            pltpu.SemaphoreType.DMA,
        ],
    )
    def k(ce_hbm, pk_hbm, re_hbm, cr_hbm, ipair_hbm, icr_hbm,
          big_out, rad_out, ip_v, ic_v, rows0_v, rows1_v, rad_v, blocks_v,
          gsem0, gsem1, fsem0, fsem1, bsem):
        wid = lax.axis_index("s") * NC + lax.axis_index("c")
        rbase = pl.multiple_of(wid * ROWS_W, ROWS_W)

        pltpu.sync_copy(
            ipair_hbm.at[pl.ds(pl.multiple_of(wid * (N_SLOT * ROWS_W),
                                              N_SLOT * ROWS_W),
                               N_SLOT * ROWS_W)], ip_v)
        pltpu.sync_copy(
            icr_hbm.at[pl.ds(pl.multiple_of(wid * (N_CR * ROWS_W),
                                            N_CR * ROWS_W),
                             N_CR * ROWS_W)], ic_v)

        HALF = ROWS_W // 2

        def gathers(table, s, h, buf, sem):
            return [
                pltpu.async_copy(
                    table.at[ip_v.at[pl.ds(
                        pl.multiple_of(s * ROWS_W + h * HALF + j * CHUNK,
                                       CHUNK),
                        CHUNK)]],
                    buf.at[pl.ds(j * CHUNK, CHUNK)],
                    sem,
                )
                for j in range(NCH // 2)
            ]

        def flush(s, h, buf, sem):
            return pltpu.async_copy(
                buf,
                big_out.at[pl.ds(
                    pl.multiple_of(s * B + rbase + h * HALF, HALF),
                    HALF), :],
                sem,
            )

        def slot_body(table, s):
            g0 = gathers(table, s, 0, rows0_v, gsem0)
            for c in g0:
                c.wait()
            f0 = flush(s, 0, rows0_v, fsem0)
            g1 = gathers(table, s, 1, rows1_v, gsem1)
            for c in g1:
                c.wait()
            f1 = flush(s, 1, rows1_v, fsem1)
            f0.wait()
            f1.wait()

        def native_half(s, h, buf):
            def grp_body(g, carry2):
                iv = ip_v[pl.ds(
                    pl.multiple_of(s * ROWS_W + h * HALF + g * 16, 16), 16)]
                cps = []
                for l in range(16):
                    off = pl.multiple_of((iv[l] >> 3) * 8, 8)
                    cps.append(pltpu.async_copy(
                        ce_hbm.at[pl.ds(off, 8), :], blocks_v.at[l], bsem))
                for c in cps:
                    c.wait()
                for l in range(16):
                    r = iv[l] & 7
                    for j in range(EMBED_DIM // 16):
                        buf[g * 16 + l, pl.ds(16 * j, 16)] = (
                            blocks_v[l, r, pl.ds(16 * j, 16)])
                return carry2

            lax.fori_loop(0, HALF // 16, grp_body, 0, unroll=False)

        def native_body(s, carry):
            native_half(s, 0, rows0_v)
            f0 = flush(s, 0, rows0_v, fsem0)
            native_half(s, 1, rows1_v)
            f1 = flush(s, 1, rows1_v, fsem1)
            f0.wait()
            f1.wait()
            return carry

        lax.fori_loop(0, 5, native_body, 0, unroll=False)

        def small_body(s, carry):
            slot_body(pk_hbm, s)
            return carry

        lax.fori_loop(5, N_CE, small_body, 0, unroll=False)
        for t in range(N_RE):
            slot_body(re_hbm, N_CE + t)

        def rad_body(gi, carry):
            cps = [
                pltpu.async_copy(
                    cr_hbm.at[ic_v.at[pl.ds(
                        pl.multiple_of(gi * ROWS_W + j * CHUNK, CHUNK),
                        CHUNK)]],
                    rad_v.at[pl.ds(j * CHUNK, CHUNK)],
                    gsem1,
                )
                for j in range(NCH)
            ]
            for c in cps:
                c.wait()
            pltpu.sync_copy(
                rad_v,
                rad_out.at[pl.ds(pl.multiple_of(gi * B + rbase, ROWS_W),
                                 ROWS_W)])
            return carry

        lax.fori_loop(0, N_CR, rad_body, 0, unroll=False)

    return k(ce_raw, pack1k, re_pack, cr_flat, idx_pair_w, idx_cr_w)


def _tc_loss_body(big_ref, par_ref, rad_ref, gam_ref, bet_ref, out_ref,
                  ssum_ref, ssq_ref, acc_ref):
    p = pl.program_id(0)
    i = pl.program_id(1)

    x = big_ref[...]
    par = par_ref[...][:, :, None]
    xs = jnp.where(par == 1, x[:, :, EMBED_DIM:], x[:, :, :EMBED_DIM])

    @pl.when(jnp.logical_and(p == 0, i == 0))
    def _init():
        ssum_ref[...] = jnp.zeros_like(ssum_ref)
        ssq_ref[...] = jnp.zeros_like(ssq_ref)
        acc_ref[0, 0] = 0.0

    @pl.when(p == 0)
    def _stats():
        ssum_ref[...] += jnp.sum(xs, axis=1)
        ssq_ref[...] += jnp.sum(xs * xs, axis=1)

    @pl.when(p == 1)
    def _loss():
        gamma = gam_ref[...].reshape(1, EMBED_DIM)
        beta = bet_ref[...].reshape(1, EMBED_DIM)
        mean = ssum_ref[...] * (1.0 / B)
        var = ssq_ref[...] * (1.0 / B) - mean * mean
        rstd = lax.rsqrt(var + BN_EPS)

        def bn(s):
            return ((xs[s] - mean[s][None, :]) * rstd[s][None, :]
                    * gamma + beta)

        ra = jnp.abs(rad_ref[...])

        def norm(v):
            return jnp.sqrt(jnp.sum(v * v, axis=1) + 1e-12)

        relu = jax.nn.relu

        y0a, y0b = bn(0), bn(1)
        t = relu(norm(y0a - y0b) + ra[0] - ra[1] - MARGIN)

        y1a, y1b, y1c = bn(2), bn(3), bn(4)
        r1a, r1b = ra[2], ra[3]
        t += (relu(norm(y1b - y1a) - (r1a + r1b) - MARGIN)
              + relu(norm(y1c - y1a) - r1a - MARGIN)
              + relu(norm(y1c - y1b) - r1b - MARGIN))

        y2a, y2c = bn(5), bn(6)
        r2a, r2c = ra[4], ra[5]
        dst = norm(y2a + xs[9] - y2c)
        t += (relu(dst + r2a - r2c - MARGIN)
              + relu(r2a + r2c - dst + MARGIN))

        y3b, y3c = bn(7), bn(8)
        t += relu(norm(y3b - xs[10] - y3c) - ra[6] - ra[7] - MARGIN)

        acc_ref[0, 0] += jnp.sum(t)

    @pl.when(jnp.logical_and(p == 1, i == NBLK - 1))
    def _fin():
        out_ref[0, 0] = acc_ref[0, 0] * (1.0 / B)


def kernel(gci0, gci1, gci2, gci3, class_embed, class_rad, rel_embed, bn_gamma, bn_beta):
    idx_slot = jnp.stack([
        gci0[:, 0], gci0[:, 1],
        gci1[:, 0], gci1[:, 1], gci1[:, 2],
        gci2[:, 0], gci2[:, 2],
        gci3[:, 1], gci3[:, 2],
        gci2[:, 1], gci3[:, 0],
    ])
    idx_pair = jnp.concatenate([idx_slot[:5], idx_slot[5:] >> 1], axis=0)
    parity = jnp.concatenate(
        [jnp.zeros((5, B), jnp.int32), (idx_slot[5:] & 1).astype(jnp.int32)],
        axis=0)

    idx_cr = jnp.stack([
        gci0[:, 0], gci0[:, 1], gci1[:, 0], gci1[:, 1],
        gci2[:, 0], gci2[:, 2], gci3[:, 1], gci3[:, 2],
    ])

    ipw = idx_pair.reshape(N_SLOT, NW, ROWS_W).transpose(1, 0, 2).reshape(-1)
    icw = idx_cr.reshape(N_CR, NW, ROWS_W).transpose(1, 0, 2).reshape(-1)

    pack1k = class_embed[:1000].reshape(500, 2 * EMBED_DIM)
    re_pack = rel_embed.reshape(500, 2 * EMBED_DIM)
    cr_flat = class_rad.reshape(-1)

    big, rad = _sc_gather(class_embed, pack1k, re_pack, cr_flat, ipw, icw)


    loss = pl.pallas_call(
        _tc_loss_body,
        out_shape=jax.ShapeDtypeStruct((1, 1), jnp.float32),
        grid=(2, NBLK),
        in_specs=[
            pl.BlockSpec((N_SLOT, BLK, CHUNK), lambda p, i: (0, i, 0)),
            pl.BlockSpec((N_SLOT, BLK), lambda p, i: (0, i)),
            pl.BlockSpec((8, BLK), lambda p, i: (0, i)),
            pl.BlockSpec((1, EMBED_DIM), lambda p, i: (0, 0)),
            pl.BlockSpec((1, EMBED_DIM), lambda p, i: (0, 0)),
        ],
        out_specs=pl.BlockSpec((1, 1), lambda p, i: (0, 0),
                               memory_space=pltpu.SMEM),
        scratch_shapes=[
            pltpu.VMEM((N_SLOT, EMBED_DIM), jnp.float32),
            pltpu.VMEM((N_SLOT, EMBED_DIM), jnp.float32),
            pltpu.SMEM((1, 1), jnp.float32),
        ],
        compiler_params=pltpu.CompilerParams(
            vmem_limit_bytes=128 * 1024 * 1024),
    )(big.reshape(N_SLOT, B, CHUNK), parity, rad.reshape(N_CR, B),
      bn_gamma.reshape(1, EMBED_DIM), bn_beta.reshape(1, EMBED_DIM))

    return jnp.reshape(loss, ())


# speedup vs baseline: 1.0261x; 1.0261x over previous
import functools

import jax
import jax.numpy as jnp
from jax import lax
from jax.experimental import pallas as pl
from jax.experimental.pallas import tpu as pltpu
from jax.experimental.pallas import tpu_sc as plsc

EMBED_DIM = 64
MARGIN = 0.1
BN_EPS = 1e-5
B = 16384

NC, NS = 2, 16
NW = NC * NS
ROWS_W = B // NW
CHUNK = 128
NCH = ROWS_W // CHUNK

N_CE = 9
N_RE = 2
N_SLOT = N_CE + N_RE
N_CR = 8

BLK = 2048
NBLK = B // BLK


def _sc_gather(ce_raw, pack1k, re_pack, idx_pair_w):
    mesh = plsc.VectorSubcoreMesh(core_axis_name="c", subcore_axis_name="s")

    @functools.partial(
        pl.kernel,
        out_type=jax.ShapeDtypeStruct((N_SLOT * B, CHUNK), jnp.float32),
        mesh=mesh,
        scratch_types=[
            pltpu.VMEM((N_SLOT * ROWS_W,), jnp.int32),
            pltpu.VMEM((ROWS_W // 2, CHUNK), jnp.float32),
            pltpu.VMEM((ROWS_W // 2, CHUNK), jnp.float32),
            pltpu.VMEM((16, 8, EMBED_DIM), jnp.float32),
            pltpu.SemaphoreType.DMA,
            pltpu.SemaphoreType.DMA,
            pltpu.SemaphoreType.DMA,
            pltpu.SemaphoreType.DMA,
            pltpu.SemaphoreType.DMA,
        ],
    )
    def k(ce_hbm, pk_hbm, re_hbm, ipair_hbm,
          big_out, ip_v, rows0_v, rows1_v, blocks_v,
          gsem0, gsem1, fsem0, fsem1, bsem):
        wid = lax.axis_index("s") * NC + lax.axis_index("c")
        rbase = pl.multiple_of(wid * ROWS_W, ROWS_W)

        pltpu.sync_copy(
            ipair_hbm.at[pl.ds(pl.multiple_of(wid * (N_SLOT * ROWS_W),
                                              N_SLOT * ROWS_W),
                               N_SLOT * ROWS_W)], ip_v)
        HALF = ROWS_W // 2

        def gathers(table, s, h, buf, sem):
            return [
                pltpu.async_copy(
                    table.at[ip_v.at[pl.ds(
                        pl.multiple_of(s * ROWS_W + h * HALF + j * CHUNK,
                                       CHUNK),
                        CHUNK)]],
                    buf.at[pl.ds(j * CHUNK, CHUNK)],
                    sem,
                )
                for j in range(NCH // 2)
            ]

        def flush(s, h, buf, sem):
            return pltpu.async_copy(
                buf,
                big_out.at[pl.ds(
                    pl.multiple_of(s * B + rbase + h * HALF, HALF),
                    HALF), :],
                sem,
            )

        def slot_body(table, s):
            g0 = gathers(table, s, 0, rows0_v, gsem0)
            for c in g0:
                c.wait()
            f0 = flush(s, 0, rows0_v, fsem0)
            g1 = gathers(table, s, 1, rows1_v, gsem1)
            for c in g1:
                c.wait()
            f1 = flush(s, 1, rows1_v, fsem1)
            f0.wait()
            f1.wait()

        def native_half(s, h, buf):
            def grp_body(g, carry2):
                iv = ip_v[pl.ds(
                    pl.multiple_of(s * ROWS_W + h * HALF + g * 16, 16), 16)]
                cps = []
                for l in range(16):
                    off = pl.multiple_of((iv[l] >> 3) * 8, 8)
                    cps.append(pltpu.async_copy(
                        ce_hbm.at[pl.ds(off, 8), :], blocks_v.at[l], bsem))
                for c in cps:
                    c.wait()
                for l in range(16):
                    r = iv[l] & 7
                    for j in range(EMBED_DIM // 16):
                        buf[g * 16 + l, pl.ds(16 * j, 16)] = (
                            blocks_v[l, r, pl.ds(16 * j, 16)])
                return carry2

            lax.fori_loop(0, HALF // 16, grp_body, 0, unroll=False)

        def native_body(s, carry):
            native_half(s, 0, rows0_v)
            f0 = flush(s, 0, rows0_v, fsem0)
            native_half(s, 1, rows1_v)
            f1 = flush(s, 1, rows1_v, fsem1)
            f0.wait()
            f1.wait()
            return carry

        lax.fori_loop(0, 5, native_body, 0, unroll=False)

        def small_body(s, carry):
            slot_body(pk_hbm, s)
            return carry

        lax.fori_loop(5, N_CE, small_body, 0, unroll=False)
        for t in range(N_RE):
            slot_body(re_hbm, N_CE + t)

    return k(ce_raw, pack1k, re_pack, idx_pair_w)


def _sc_rad(cr_flat, idx_cr_w):
    mesh = plsc.VectorSubcoreMesh(core_axis_name="c", subcore_axis_name="s")

    @functools.partial(
        pl.kernel,
        out_type=jax.ShapeDtypeStruct((N_CR * B,), jnp.float32),
        mesh=mesh,
        scratch_types=[
            pltpu.VMEM((N_CR * ROWS_W,), jnp.int32),
            pltpu.VMEM((ROWS_W,), jnp.float32),
            pltpu.SemaphoreType.DMA,
        ],
    )
    def k(cr_hbm, icr_hbm, rad_out, ic_v, rad_v, gsem):
        wid = lax.axis_index("s") * NC + lax.axis_index("c")
        rbase = pl.multiple_of(wid * ROWS_W, ROWS_W)
        pltpu.sync_copy(
            icr_hbm.at[pl.ds(pl.multiple_of(wid * (N_CR * ROWS_W),
                                            N_CR * ROWS_W),
                             N_CR * ROWS_W)], ic_v)

        def rad_body(gi, carry):
            cps = [
                pltpu.async_copy(
                    cr_hbm.at[ic_v.at[pl.ds(
                        pl.multiple_of(gi * ROWS_W + j * CHUNK, CHUNK),
                        CHUNK)]],
                    rad_v.at[pl.ds(j * CHUNK, CHUNK)],
                    gsem,
                )
                for j in range(NCH)
            ]
            for c in cps:
                c.wait()
            pltpu.sync_copy(
                rad_v,
                rad_out.at[pl.ds(pl.multiple_of(gi * B + rbase, ROWS_W),
                                 ROWS_W)])
            return carry

        lax.fori_loop(0, N_CR, rad_body, 0, unroll=False)

    return k(cr_flat, idx_cr_w)


def _tc_loss_body(big_ref, par_ref, rad_ref, gam_ref, bet_ref, out_ref,
                  ssum_ref, ssq_ref, acc_ref):
    p = pl.program_id(0)
    i = pl.program_id(1)

    x = big_ref[...]
    par = par_ref[...][:, :, None]
    xs = jnp.where(par == 1, x[:, :, EMBED_DIM:], x[:, :, :EMBED_DIM])

    @pl.when(jnp.logical_and(p == 0, i == 0))
    def _init():
        ssum_ref[...] = jnp.zeros_like(ssum_ref)
        ssq_ref[...] = jnp.zeros_like(ssq_ref)
        acc_ref[0, 0] = 0.0

    @pl.when(p == 0)
    def _stats():
        ssum_ref[...] += jnp.sum(xs, axis=1)
        ssq_ref[...] += jnp.sum(xs * xs, axis=1)

    @pl.when(p == 1)
    def _loss():
        gamma = gam_ref[...].reshape(1, EMBED_DIM)
        beta = bet_ref[...].reshape(1, EMBED_DIM)
        mean = ssum_ref[...] * (1.0 / B)
        var = ssq_ref[...] * (1.0 / B) - mean * mean
        rstd = lax.rsqrt(var + BN_EPS)

        def bn(s):
            return ((xs[s] - mean[s][None, :]) * rstd[s][None, :]
                    * gamma + beta)

        ra = jnp.abs(rad_ref[...])

        def norm(v):
            return jnp.sqrt(jnp.sum(v * v, axis=1) + 1e-12)

        relu = jax.nn.relu

        y0a, y0b = bn(0), bn(1)
        t = relu(norm(y0a - y0b) + ra[0] - ra[1] - MARGIN)

        y1a, y1b, y1c = bn(2), bn(3), bn(4)
        r1a, r1b = ra[2], ra[3]
        t += (relu(norm(y1b - y1a) - (r1a + r1b) - MARGIN)
              + relu(norm(y1c - y1a) - r1a - MARGIN)
              + relu(norm(y1c - y1b) - r1b - MARGIN))

        y2a, y2c = bn(5), bn(6)
        r2a, r2c = ra[4], ra[5]
        dst = norm(y2a + xs[9] - y2c)
        t += (relu(dst + r2a - r2c - MARGIN)
              + relu(r2a + r2c - dst + MARGIN))

        y3b, y3c = bn(7), bn(8)
        t += relu(norm(y3b - xs[10] - y3c) - ra[6] - ra[7] - MARGIN)

        acc_ref[0, 0] += jnp.sum(t)

    @pl.when(jnp.logical_and(p == 1, i == NBLK - 1))
    def _fin():
        out_ref[0, 0] = acc_ref[0, 0] * (1.0 / B)


def kernel(gci0, gci1, gci2, gci3, class_embed, class_rad, rel_embed, bn_gamma, bn_beta):
    idx_slot = jnp.stack([
        gci0[:, 0], gci0[:, 1],
        gci1[:, 0], gci1[:, 1], gci1[:, 2],
        gci2[:, 0], gci2[:, 2],
        gci3[:, 1], gci3[:, 2],
        gci2[:, 1], gci3[:, 0],
    ])
    idx_pair = jnp.concatenate([idx_slot[:5], idx_slot[5:] >> 1], axis=0)
    parity = jnp.concatenate(
        [jnp.zeros((5, B), jnp.int32), (idx_slot[5:] & 1).astype(jnp.int32)],
        axis=0)

    idx_cr = jnp.stack([
        gci0[:, 0], gci0[:, 1], gci1[:, 0], gci1[:, 1],
        gci2[:, 0], gci2[:, 2], gci3[:, 1], gci3[:, 2],
    ])

    ipw = idx_pair.reshape(N_SLOT, NW, ROWS_W).transpose(1, 0, 2).reshape(-1)
    icw = idx_cr.reshape(N_CR, NW, ROWS_W).transpose(1, 0, 2).reshape(-1)

    pack1k = class_embed[:1000].reshape(500, 2 * EMBED_DIM)
    re_pack = rel_embed.reshape(500, 2 * EMBED_DIM)
    cr_flat = class_rad.reshape(-1)

    big = _sc_gather(class_embed, pack1k, re_pack, ipw)
    rad = _sc_rad(cr_flat, icw)


    loss = pl.pallas_call(
        _tc_loss_body,
        out_shape=jax.ShapeDtypeStruct((1, 1), jnp.float32),
        grid=(2, NBLK),
        in_specs=[
            pl.BlockSpec((N_SLOT, BLK, CHUNK), lambda p, i: (0, i, 0)),
            pl.BlockSpec((N_SLOT, BLK), lambda p, i: (0, i)),
            pl.BlockSpec((8, BLK), lambda p, i: (0, i)),
            pl.BlockSpec((1, EMBED_DIM), lambda p, i: (0, 0)),
            pl.BlockSpec((1, EMBED_DIM), lambda p, i: (0, 0)),
        ],
        out_specs=pl.BlockSpec((1, 1), lambda p, i: (0, 0),
                               memory_space=pltpu.SMEM),
        scratch_shapes=[
            pltpu.VMEM((N_SLOT, EMBED_DIM), jnp.float32),
            pltpu.VMEM((N_SLOT, EMBED_DIM), jnp.float32),
            pltpu.SMEM((1, 1), jnp.float32),
        ],
        compiler_params=pltpu.CompilerParams(
            vmem_limit_bytes=128 * 1024 * 1024),
    )(big.reshape(N_SLOT, B, CHUNK), parity, rad.reshape(N_CR, B),
      bn_gamma.reshape(1, EMBED_DIM), bn_beta.reshape(1, EMBED_DIM))

    return jnp.reshape(loss, ())


# speedup vs baseline: 1.1220x; 1.0935x over previous
import functools

import jax
import jax.numpy as jnp
from jax import lax
from jax.experimental import pallas as pl
from jax.experimental.pallas import tpu as pltpu
from jax.experimental.pallas import tpu_sc as plsc

EMBED_DIM = 64
MARGIN = 0.1
BN_EPS = 1e-5
B = 16384

NC, NS = 2, 16
NW = NC * NS
ROWS_W = B // NW
CHUNK = 128
NCH = ROWS_W // CHUNK

N_CE = 9
N_RE = 2
N_SLOT = N_CE + N_RE
N_CR = 8

BLK = 2048
NBLK = B // BLK


def _sc_gather(ce_raw, pack1k, re_pack, idx_pair_w):
    mesh = plsc.VectorSubcoreMesh(core_axis_name="c", subcore_axis_name="s")

    @functools.partial(
        pl.kernel,
        out_type=jax.ShapeDtypeStruct((N_SLOT * B, CHUNK), jnp.float32),
        mesh=mesh,
        scratch_types=[
            pltpu.VMEM((N_SLOT * ROWS_W,), jnp.int32),
            pltpu.VMEM((ROWS_W // 2, CHUNK), jnp.float32),
            pltpu.VMEM((ROWS_W // 2, CHUNK), jnp.float32),
            pltpu.VMEM((16, 8, EMBED_DIM), jnp.float32),
            pltpu.VMEM((16, 8, EMBED_DIM), jnp.float32),
            pltpu.SemaphoreType.DMA,
            pltpu.SemaphoreType.DMA,
            pltpu.SemaphoreType.DMA,
            pltpu.SemaphoreType.DMA,
            pltpu.SemaphoreType.DMA,
            pltpu.SemaphoreType.DMA,
        ],
    )
    def k(ce_hbm, pk_hbm, re_hbm, ipair_hbm,
          big_out, ip_v, rows0_v, rows1_v, blocksA_v, blocksB_v,
          gsem0, gsem1, fsem0, fsem1, bsemA, bsemB):
        wid = lax.axis_index("s") * NC + lax.axis_index("c")
        rbase = pl.multiple_of(wid * ROWS_W, ROWS_W)

        pltpu.sync_copy(
            ipair_hbm.at[pl.ds(pl.multiple_of(wid * (N_SLOT * ROWS_W),
                                              N_SLOT * ROWS_W),
                               N_SLOT * ROWS_W)], ip_v)
        HALF = ROWS_W // 2

        def gathers(table, s, h, buf, sem):
            return [
                pltpu.async_copy(
                    table.at[ip_v.at[pl.ds(
                        pl.multiple_of(s * ROWS_W + h * HALF + j * CHUNK,
                                       CHUNK),
                        CHUNK)]],
                    buf.at[pl.ds(j * CHUNK, CHUNK)],
                    sem,
                )
                for j in range(NCH // 2)
            ]

        def flush(s, h, buf, sem):
            return pltpu.async_copy(
                buf,
                big_out.at[pl.ds(
                    pl.multiple_of(s * B + rbase + h * HALF, HALF),
                    HALF), :],
                sem,
            )

        def slot_body(table, s):
            g0 = gathers(table, s, 0, rows0_v, gsem0)
            for c in g0:
                c.wait()
            f0 = flush(s, 0, rows0_v, fsem0)
            g1 = gathers(table, s, 1, rows1_v, gsem1)
            for c in g1:
                c.wait()
            f1 = flush(s, 1, rows1_v, fsem1)
            f0.wait()
            f1.wait()

        NG = HALF // 16

        def load_iv(s, h, g):
            return ip_v[pl.ds(
                pl.multiple_of(s * ROWS_W + h * HALF + g * 16, 16), 16)]

        def issue(iv, blocks, sem):
            for l in range(16):
                off = pl.multiple_of((iv[l] >> 3) * 8, 8)
                pltpu.async_copy(ce_hbm.at[pl.ds(off, 8), :],
                                 blocks.at[l], sem)

        def drain16(blocks, sem):
            for l in range(16):
                pltpu.make_async_copy(ce_hbm.at[pl.ds(0, 8), :],
                                      blocks.at[l], sem).wait()

        def extract(iv, g, blocks, buf):
            for l in range(16):
                r = iv[l] & 7
                for j in range(EMBED_DIM // 16):
                    buf[g * 16 + l, pl.ds(16 * j, 16)] = (
                        blocks[l, r, pl.ds(16 * j, 16)])

        def native_half(s, h, buf):
            issue(load_iv(s, h, 0), blocksA_v, bsemA)

            def pair_body(g2, carry2):
                ga = 2 * g2
                ivB = load_iv(s, h, ga + 1)
                issue(ivB, blocksB_v, bsemB)
                ivA = load_iv(s, h, ga)
                drain16(blocksA_v, bsemA)
                extract(ivA, ga, blocksA_v, buf)

                @pl.when(ga + 2 < NG)
                def _():
                    issue(load_iv(s, h, ga + 2), blocksA_v, bsemA)

                drain16(blocksB_v, bsemB)
                extract(ivB, ga + 1, blocksB_v, buf)
                return carry2

            lax.fori_loop(0, NG // 2, pair_body, 0, unroll=False)

        def native_body(s, carry):
            native_half(s, 0, rows0_v)
            f0 = flush(s, 0, rows0_v, fsem0)
            native_half(s, 1, rows1_v)
            f1 = flush(s, 1, rows1_v, fsem1)
            f0.wait()
            f1.wait()
            return carry

        lax.fori_loop(0, 5, native_body, 0, unroll=False)

        def small_body(s, carry):
            slot_body(pk_hbm, s)
            return carry

        lax.fori_loop(5, N_CE, small_body, 0, unroll=False)
        for t in range(N_RE):
            slot_body(re_hbm, N_CE + t)

    return k(ce_raw, pack1k, re_pack, idx_pair_w)


def _sc_rad(cr_flat, idx_cr_w):
    mesh = plsc.VectorSubcoreMesh(core_axis_name="c", subcore_axis_name="s")

    @functools.partial(
        pl.kernel,
        out_type=jax.ShapeDtypeStruct((N_CR * B,), jnp.float32),
        mesh=mesh,
        scratch_types=[
            pltpu.VMEM((N_CR * ROWS_W,), jnp.int32),
            pltpu.VMEM((ROWS_W,), jnp.float32),
            pltpu.SemaphoreType.DMA,
        ],
    )
    def k(cr_hbm, icr_hbm, rad_out, ic_v, rad_v, gsem):
        wid = lax.axis_index("s") * NC + lax.axis_index("c")
        rbase = pl.multiple_of(wid * ROWS_W, ROWS_W)
        pltpu.sync_copy(
            icr_hbm.at[pl.ds(pl.multiple_of(wid * (N_CR * ROWS_W),
                                            N_CR * ROWS_W),
                             N_CR * ROWS_W)], ic_v)

        def rad_body(gi, carry):
            cps = [
                pltpu.async_copy(
                    cr_hbm.at[ic_v.at[pl.ds(
                        pl.multiple_of(gi * ROWS_W + j * CHUNK, CHUNK),
                        CHUNK)]],
                    rad_v.at[pl.ds(j * CHUNK, CHUNK)],
                    gsem,
                )
                for j in range(NCH)
            ]
            for c in cps:
                c.wait()
            pltpu.sync_copy(
                rad_v,
                rad_out.at[pl.ds(pl.multiple_of(gi * B + rbase, ROWS_W),
                                 ROWS_W)])
            return carry

        lax.fori_loop(0, N_CR, rad_body, 0, unroll=False)

    return k(cr_flat, idx_cr_w)


def _tc_loss_body(big_ref, par_ref, rad_ref, gam_ref, bet_ref, out_ref,
                  ssum_ref, ssq_ref, acc_ref):
    p = pl.program_id(0)
    i = pl.program_id(1)

    x = big_ref[...]
    par = par_ref[...][:, :, None]
    xs = jnp.where(par == 1, x[:, :, EMBED_DIM:], x[:, :, :EMBED_DIM])

    @pl.when(jnp.logical_and(p == 0, i == 0))
    def _init():
        ssum_ref[...] = jnp.zeros_like(ssum_ref)
        ssq_ref[...] = jnp.zeros_like(ssq_ref)
        acc_ref[0, 0] = 0.0

    @pl.when(p == 0)
    def _stats():
        ssum_ref[...] += jnp.sum(xs, axis=1)
        ssq_ref[...] += jnp.sum(xs * xs, axis=1)

    @pl.when(p == 1)
    def _loss():
        gamma = gam_ref[...].reshape(1, EMBED_DIM)
        beta = bet_ref[...].reshape(1, EMBED_DIM)
        mean = ssum_ref[...] * (1.0 / B)
        var = ssq_ref[...] * (1.0 / B) - mean * mean
        rstd = lax.rsqrt(var + BN_EPS)

        def bn(s):
            return ((xs[s] - mean[s][None, :]) * rstd[s][None, :]
                    * gamma + beta)

        ra = jnp.abs(rad_ref[...])

        def norm(v):
            return jnp.sqrt(jnp.sum(v * v, axis=1) + 1e-12)

        relu = jax.nn.relu

        y0a, y0b = bn(0), bn(1)
        t = relu(norm(y0a - y0b) + ra[0] - ra[1] - MARGIN)

        y1a, y1b, y1c = bn(2), bn(3), bn(4)
        r1a, r1b = ra[2], ra[3]
        t += (relu(norm(y1b - y1a) - (r1a + r1b) - MARGIN)
              + relu(norm(y1c - y1a) - r1a - MARGIN)
              + relu(norm(y1c - y1b) - r1b - MARGIN))

        y2a, y2c = bn(5), bn(6)
        r2a, r2c = ra[4], ra[5]
        dst = norm(y2a + xs[9] - y2c)
        t += (relu(dst + r2a - r2c - MARGIN)
              + relu(r2a + r2c - dst + MARGIN))

        y3b, y3c = bn(7), bn(8)
        t += relu(norm(y3b - xs[10] - y3c) - ra[6] - ra[7] - MARGIN)

        acc_ref[0, 0] += jnp.sum(t)

    @pl.when(jnp.logical_and(p == 1, i == NBLK - 1))
    def _fin():
        out_ref[0, 0] = acc_ref[0, 0] * (1.0 / B)


def kernel(gci0, gci1, gci2, gci3, class_embed, class_rad, rel_embed, bn_gamma, bn_beta):
    idx_slot = jnp.stack([
        gci0[:, 0], gci0[:, 1],
        gci1[:, 0], gci1[:, 1], gci1[:, 2],
        gci2[:, 0], gci2[:, 2],
        gci3[:, 1], gci3[:, 2],
        gci2[:, 1], gci3[:, 0],
    ])
    idx_pair = jnp.concatenate([idx_slot[:5], idx_slot[5:] >> 1], axis=0)
    parity = jnp.concatenate(
        [jnp.zeros((5, B), jnp.int32), (idx_slot[5:] & 1).astype(jnp.int32)],
        axis=0)

    idx_cr = jnp.stack([
        gci0[:, 0], gci0[:, 1], gci1[:, 0], gci1[:, 1],
        gci2[:, 0], gci2[:, 2], gci3[:, 1], gci3[:, 2],
    ])

    ipw = idx_pair.reshape(N_SLOT, NW, ROWS_W).transpose(1, 0, 2).reshape(-1)
    icw = idx_cr.reshape(N_CR, NW, ROWS_W).transpose(1, 0, 2).reshape(-1)

    pack1k = class_embed[:1000].reshape(500, 2 * EMBED_DIM)
    re_pack = rel_embed.reshape(500, 2 * EMBED_DIM)
    cr_flat = class_rad.reshape(-1)

    big = _sc_gather(class_embed, pack1k, re_pack, ipw)
    rad = _sc_rad(cr_flat, icw)


    loss = pl.pallas_call(
        _tc_loss_body,
        out_shape=jax.ShapeDtypeStruct((1, 1), jnp.float32),
        grid=(2, NBLK),
        in_specs=[
            pl.BlockSpec((N_SLOT, BLK, CHUNK), lambda p, i: (0, i, 0)),
            pl.BlockSpec((N_SLOT, BLK), lambda p, i: (0, i)),
            pl.BlockSpec((8, BLK), lambda p, i: (0, i)),
            pl.BlockSpec((1, EMBED_DIM), lambda p, i: (0, 0)),
            pl.BlockSpec((1, EMBED_DIM), lambda p, i: (0, 0)),
        ],
        out_specs=pl.BlockSpec((1, 1), lambda p, i: (0, 0),
                               memory_space=pltpu.SMEM),
        scratch_shapes=[
            pltpu.VMEM((N_SLOT, EMBED_DIM), jnp.float32),
            pltpu.VMEM((N_SLOT, EMBED_DIM), jnp.float32),
            pltpu.SMEM((1, 1), jnp.float32),
        ],
        compiler_params=pltpu.CompilerParams(
            vmem_limit_bytes=128 * 1024 * 1024),
    )(big.reshape(N_SLOT, B, CHUNK), parity, rad.reshape(N_CR, B),
      bn_gamma.reshape(1, EMBED_DIM), bn_beta.reshape(1, EMBED_DIM))

    return jnp.reshape(loss, ())
